# Initial kernel scaffold; baseline (speedup 1.0000x reference)
import jax
import jax.numpy as jnp
from jax.experimental import pallas as pl


def kernel(s, t, edge_index, lins_W, lins_b, lint_W, lint_b, conv_w):
    raise NotImplementedError("write your pallas kernel here")


# speedup vs baseline: 11.5941x; 11.5941x over previous
import functools

import jax
import jax.numpy as jnp
from jax import lax
from jax.experimental import pallas as pl
from jax.experimental.pallas import tpu as pltpu, tpu_sc as plsc

N = 10000
D = 128
NP = 10240
NPAD_ROWS = 8
NC, NS, L = 2, 16, 16
CB = 128
C = 160
EPT = C * CB
EP = NS * EPT
RPT = NP // NS

_mesh = plsc.VectorSubcoreMesh(
    core_axis_name="c", subcore_axis_name="s", num_cores=NC, num_subcores=NS)


@functools.partial(
    pl.kernel,
    out_type=jax.ShapeDtypeStruct((NC * NP,), jnp.float32),
    mesh=_mesh,
    scratch_types=[
        pltpu.VMEM((C, CB), jnp.int32),
        pltpu.VMEM((CB,), jnp.float32),
        pltpu.VMEM((RPT,), jnp.float32),
        pltpu.VMEM_SHARED((NP,), jnp.float32),
        pltpu.SemaphoreType.DMA,
    ],
)
def _sc_degrees(sidx_all, degs, idx_v, ones_v, z_v, hist, sem):
    cid = lax.axis_index("c")
    sid = lax.axis_index("s")
    for i in range(CB // L):
        ones_v[pl.ds(i * L, L)] = jnp.ones((L,), jnp.float32)
    for i in range(RPT // L):
        z_v[pl.ds(i * L, L)] = jnp.zeros((L,), jnp.float32)
    pltpu.sync_copy(z_v, hist.at[pl.ds(sid * RPT, RPT)])
    pltpu.sync_copy(sidx_all.at[pl.ds((cid * NS + sid) * C, C)], idx_v)
    plsc.subcore_barrier()

    @pl.loop(0, C)
    def _(c):
        pltpu.sync_copy(ones_v, hist.at[idx_v.at[c]], add=True)

    plsc.subcore_barrier()
    pltpu.sync_copy(hist.at[pl.ds(sid * RPT, RPT)],
                    degs.at[pl.ds((1 - cid) * NP + sid * RPT, RPT)])


HD = D // 2


@functools.partial(
    pl.kernel,
    out_type=jax.ShapeDtypeStruct((2, NC * NP, HD), jnp.float32),
    mesh=_mesh,
    scratch_types=[
        pltpu.VMEM((C, CB), jnp.int32),
        pltpu.VMEM((C, CB), jnp.int32),
        pltpu.VMEM((CB, HD), jnp.float32),
        pltpu.VMEM_SHARED((NP, HD), jnp.float32),
        pltpu.SemaphoreType.DMA,
    ],
    compiler_params=pltpu.CompilerParams(use_tc_tiling_on_sc=False),
)
def _sc_round(g2, gidx_all, sidx_all, acc2, gidx, sidx, rows, acc, sem):
    cid = lax.axis_index("c")
    sid = lax.axis_index("s")
    rsl = pl.ds(sid * RPT, RPT)
    csl = pl.ds((cid * NS + sid) * C, C)

    pltpu.sync_copy(gidx_all.at[csl], gidx)
    pltpu.sync_copy(sidx_all.at[csl], sidx)

    for hh in range(2):
        src = g2.at[hh]
        pltpu.sync_copy(src.at[pl.ds(cid * NP + sid * RPT, RPT)],
                        acc.at[rsl])
        plsc.subcore_barrier()

        @pl.loop(0, C)
        def _(c):
            pltpu.async_copy(src.at[gidx.at[c]], rows, sem).wait()
            pltpu.sync_copy(rows, acc.at[sidx.at[c]], add=True)

        plsc.subcore_barrier()
        pltpu.sync_copy(
            acc.at[rsl],
            acc2.at[hh].at[pl.ds((1 - cid) * NP + sid * RPT, RPT)])
        plsc.subcore_barrier()


_BR = 1280
_HGRID = NP // _BR
_GRID = NC * _HGRID

_dn = (((1,), (1,)), ((), ()))


def _dot_t(x, w):
    return lax.dot_general(x, w, dimension_numbers=_dn,
                           preferred_element_type=jnp.float32,
                           precision=lax.Precision.HIGHEST)


def _rows_spec():
    return pl.BlockSpec((_BR, D), lambda i: (i, 0))


def _colv_spec():
    return pl.BlockSpec((_BR, 1), lambda i: (i, 0))


def _gsplit_spec():
    return pl.BlockSpec((2, _BR, HD), lambda i: (0, i, 0))


def _write_split(g_ref, full):
    g_ref[0] = full[:, :HD]
    g_ref[1] = full[:, HD:]


def _mlp_body(u_ref, w_ref, b_ref, deg_ref, uh_ref, g_ref, sc_ref):
    x = u_ref[...]
    h = jnp.maximum(_dot_t(x, w_ref[0, 0]) + b_ref[0, 0][None, :], 0.0)
    xh = _dot_t(h, w_ref[0, 1]) + b_ref[0, 1][None, :]
    sc = lax.rsqrt(deg_ref[...] + 1.0)
    uh_ref[...] = xh
    sc_ref[...] = sc
    _write_split(g_ref, sc * xh)


def _mlp(u, wcat, bcat, deg2):
    out_nd = jax.ShapeDtypeStruct((NC * NP, D), jnp.float32)
    out_g = jax.ShapeDtypeStruct((2, NC * NP, HD), jnp.float32)
    out_n1 = jax.ShapeDtypeStruct((NC * NP, 1), jnp.float32)
    return pl.pallas_call(
        _mlp_body,
        grid=(_GRID,),
        in_specs=[
            _rows_spec(),
            pl.BlockSpec((1, 2, D, D), lambda i: (i // _HGRID, 0, 0, 0)),
            pl.BlockSpec((1, 2, D), lambda i: (i // _HGRID, 0, 0)),
            _colv_spec(),
        ],
        out_specs=[_rows_spec(), _gsplit_spec(), _colv_spec()],
        out_shape=[out_nd, out_g, out_n1],
    )(u, wcat, bcat, deg2)


def _combine_body(with_g, u_ref, acc_ref, sc_ref, cw_ref, *outs):
    sc = sc_ref[...]
    accf = jnp.concatenate([acc_ref[0], acc_ref[1]], axis=1)
    u1 = u_ref[...] + cw_ref[0, 0, 0] * sc * accf
    outs[0][...] = u1
    if with_g:
        _write_split(outs[1], sc * u1)


def _combine(u, acc2, scv, cvec, with_g):
    out_nd = jax.ShapeDtypeStruct((NC * NP, D), jnp.float32)
    out_g = jax.ShapeDtypeStruct((2, NC * NP, HD), jnp.float32)
    out_specs = [_rows_spec()]
    out_shape = [out_nd]
    if with_g:
        out_specs.append(_gsplit_spec())
        out_shape.append(out_g)
    return pl.pallas_call(
        functools.partial(_combine_body, with_g),
        grid=(_GRID,),
        in_specs=[
            _rows_spec(),
            _gsplit_spec(),
            _colv_spec(),
            pl.BlockSpec((1, 1, 1), lambda i: (i // _HGRID, 0, 0)),
        ],
        out_specs=out_specs,
        out_shape=out_shape,
    )(u, acc2, scv, cvec)


@jax.jit
def kernel(s, t, edge_index, lins_W, lins_b, lint_W, lint_b, conv_w):
    row = edge_index[0]
    col = edge_index[1]
    e = row.shape[0]
    pad = EP - e
    padv = N + (jnp.arange(pad, dtype=jnp.int32) % NPAD_ROWS)
    rowp = jnp.concatenate([row, padv])
    colp = jnp.concatenate([col, padv])
    gidx_all = jnp.concatenate([colp, rowp + NP]).reshape(NC * NS * C, CB)
    sidx_all = jnp.concatenate([rowp, colp]).reshape(NC * NS * C, CB)

    zpad = jnp.zeros((NP - N, D), jnp.float32)
    u0 = jnp.concatenate([t, zpad, s, zpad])

    wcat = jnp.stack([lint_W, lins_W])
    bcat = jnp.stack([lint_b, lins_b])

    degs = _sc_degrees(sidx_all)
    uh, g0, scv = _mlp(u0, wcat, bcat, degs.reshape(NC * NP, 1))

    acc1 = _sc_round(g0, gidx_all, sidx_all)
    cv0 = conv_w[0, ::-1].reshape(2, 1, 1)
    u1, g1 = _combine(uh, acc1, scv, cv0, True)

    acc2 = _sc_round(g1, gidx_all, sidx_all)
    cv1 = conv_w[1, ::-1].reshape(2, 1, 1)
    u2 = _combine(u1, acc2, scv, cv1, False)[0]

    return (u2[NP:NP + N], u2[:N])


# speedup vs baseline: 17.2080x; 1.4842x over previous
import functools

import jax
import jax.numpy as jnp
from jax import lax
from jax.experimental import pallas as pl
from jax.experimental.pallas import tpu as pltpu, tpu_sc as plsc

N = 10000
D = 128
NP = 10240
NPAD_ROWS = 8
NC, NS, L = 2, 16, 16
CB = 128
C = 160
EPT = C * CB
EP = NS * EPT
RPT = NP // NS

_mesh = plsc.VectorSubcoreMesh(
    core_axis_name="c", subcore_axis_name="s", num_cores=NC, num_subcores=NS)


@functools.partial(
    pl.kernel,
    out_type=jax.ShapeDtypeStruct((NC * NP,), jnp.float32),
    mesh=_mesh,
    scratch_types=[
        pltpu.VMEM((C, CB), jnp.int32),
        pltpu.VMEM((CB,), jnp.float32),
        pltpu.VMEM((RPT,), jnp.float32),
        pltpu.VMEM_SHARED((NP,), jnp.float32),
        pltpu.SemaphoreType.DMA,
    ],
)
def _sc_degrees(sidx_all, degs, idx_v, ones_v, z_v, hist, sem):
    cid = lax.axis_index("c")
    sid = lax.axis_index("s")
    for i in range(CB // L):
        ones_v[pl.ds(i * L, L)] = jnp.ones((L,), jnp.float32)
    for i in range(RPT // L):
        z_v[pl.ds(i * L, L)] = jnp.zeros((L,), jnp.float32)
    pltpu.sync_copy(z_v, hist.at[pl.ds(sid * RPT, RPT)])
    pltpu.sync_copy(sidx_all.at[pl.ds((cid * NS + sid) * C, C)], idx_v)
    plsc.subcore_barrier()

    @pl.loop(0, C)
    def _(c):
        pltpu.sync_copy(ones_v, hist.at[idx_v.at[c]], add=True)

    plsc.subcore_barrier()
    pltpu.sync_copy(hist.at[pl.ds(sid * RPT, RPT)],
                    degs.at[pl.ds((1 - cid) * NP + sid * RPT, RPT)])


HD = D // 2


@functools.partial(
    pl.kernel,
    out_type=jax.ShapeDtypeStruct((2, NC * NP, HD), jnp.float32),
    mesh=_mesh,
    scratch_types=[
        pltpu.VMEM((C, CB), jnp.int32),
        pltpu.VMEM((C, CB), jnp.int32),
        pltpu.VMEM((CB, HD), jnp.float32),
        pltpu.VMEM((CB, HD), jnp.float32),
        pltpu.VMEM_SHARED((NP, HD), jnp.float32),
        pltpu.SemaphoreType.DMA,
        pltpu.SemaphoreType.DMA,
    ],
    compiler_params=pltpu.CompilerParams(use_tc_tiling_on_sc=False),
)
def _sc_round(g2, gidx_all, sidx_all, acc2,
              gidx, sidx, rows0, rows1, acc, sem0, sem1):
    cid = lax.axis_index("c")
    sid = lax.axis_index("s")
    rsl = pl.ds(sid * RPT, RPT)
    csl = pl.ds((cid * NS + sid) * C, C)

    pltpu.sync_copy(gidx_all.at[csl], gidx)
    pltpu.sync_copy(sidx_all.at[csl], sidx)

    for hh in range(2):
        src = g2.at[hh]
        pltpu.sync_copy(src.at[pl.ds(cid * NP + sid * RPT, RPT)],
                        acc.at[rsl])
        plsc.subcore_barrier()

        pltpu.async_copy(src.at[gidx.at[0]], rows0, sem0)

        @pl.loop(0, C, step=2)
        def _(c):
            pltpu.async_copy(src.at[gidx.at[c + 1]], rows1, sem1)
            pltpu.make_async_copy(src.at[pl.ds(0, CB)], rows0, sem0).wait()
            pltpu.sync_copy(rows0, acc.at[sidx.at[c]], add=True)

            @pl.when(c + 2 < C)
            def _():
                pltpu.async_copy(src.at[gidx.at[c + 2]], rows0, sem0)

            pltpu.make_async_copy(src.at[pl.ds(0, CB)], rows1, sem1).wait()
            pltpu.sync_copy(rows1, acc.at[sidx.at[c + 1]], add=True)

        plsc.subcore_barrier()
        pltpu.sync_copy(
            acc.at[rsl],
            acc2.at[hh].at[pl.ds((1 - cid) * NP + sid * RPT, RPT)])
        plsc.subcore_barrier()


_BR = 1280
_HGRID = NP // _BR
_GRID = NC * _HGRID

_dn = (((1,), (1,)), ((), ()))


def _dot_t(x, w):
    return lax.dot_general(x, w, dimension_numbers=_dn,
                           preferred_element_type=jnp.float32,
                           precision=lax.Precision.HIGHEST)


def _rows_spec():
    return pl.BlockSpec((_BR, D), lambda i: (i, 0))


def _colv_spec():
    return pl.BlockSpec((_BR, 1), lambda i: (i, 0))


def _gsplit_spec():
    return pl.BlockSpec((2, _BR, HD), lambda i: (0, i, 0))


def _write_split(g_ref, full):
    g_ref[0] = full[:, :HD]
    g_ref[1] = full[:, HD:]


def _mlp_body(u_ref, w_ref, b_ref, deg_ref, uh_ref, g_ref, sc_ref):
    x = u_ref[...]
    h = jnp.maximum(_dot_t(x, w_ref[0, 0]) + b_ref[0, 0][None, :], 0.0)
    xh = _dot_t(h, w_ref[0, 1]) + b_ref[0, 1][None, :]
    sc = lax.rsqrt(deg_ref[...] + 1.0)
    uh_ref[...] = xh
    sc_ref[...] = sc
    _write_split(g_ref, sc * xh)


def _mlp(u, wcat, bcat, deg2):
    out_nd = jax.ShapeDtypeStruct((NC * NP, D), jnp.float32)
    out_g = jax.ShapeDtypeStruct((2, NC * NP, HD), jnp.float32)
    out_n1 = jax.ShapeDtypeStruct((NC * NP, 1), jnp.float32)
    return pl.pallas_call(
        _mlp_body,
        grid=(_GRID,),
        in_specs=[
            _rows_spec(),
            pl.BlockSpec((1, 2, D, D), lambda i: (i // _HGRID, 0, 0, 0)),
            pl.BlockSpec((1, 2, D), lambda i: (i // _HGRID, 0, 0)),
            _colv_spec(),
        ],
        out_specs=[_rows_spec(), _gsplit_spec(), _colv_spec()],
        out_shape=[out_nd, out_g, out_n1],
    )(u, wcat, bcat, deg2)


def _combine_body(with_g, u_ref, acc_ref, sc_ref, cw_ref, *outs):
    sc = sc_ref[...]
    accf = jnp.concatenate([acc_ref[0], acc_ref[1]], axis=1)
    u1 = u_ref[...] + cw_ref[0, 0, 0] * sc * accf
    outs[0][...] = u1
    if with_g:
        _write_split(outs[1], sc * u1)


def _combine(u, acc2, scv, cvec, with_g):
    out_nd = jax.ShapeDtypeStruct((NC * NP, D), jnp.float32)
    out_g = jax.ShapeDtypeStruct((2, NC * NP, HD), jnp.float32)
    out_specs = [_rows_spec()]
    out_shape = [out_nd]
    if with_g:
        out_specs.append(_gsplit_spec())
        out_shape.append(out_g)
    return pl.pallas_call(
        functools.partial(_combine_body, with_g),
        grid=(_GRID,),
        in_specs=[
            _rows_spec(),
            _gsplit_spec(),
            _colv_spec(),
            pl.BlockSpec((1, 1, 1), lambda i: (i // _HGRID, 0, 0)),
        ],
        out_specs=out_specs,
        out_shape=out_shape,
    )(u, acc2, scv, cvec)


@jax.jit
def kernel(s, t, edge_index, lins_W, lins_b, lint_W, lint_b, conv_w):
    row = edge_index[0]
    col = edge_index[1]
    e = row.shape[0]
    pad = EP - e
    padv = N + (jnp.arange(pad, dtype=jnp.int32) % NPAD_ROWS)
    rowp = jnp.concatenate([row, padv])
    colp = jnp.concatenate([col, padv])
    gidx_all = jnp.concatenate([colp, rowp + NP]).reshape(NC * NS * C, CB)
    sidx_all = jnp.concatenate([rowp, colp]).reshape(NC * NS * C, CB)

    zpad = jnp.zeros((NP - N, D), jnp.float32)
    u0 = jnp.concatenate([t, zpad, s, zpad])

    wcat = jnp.stack([lint_W, lins_W])
    bcat = jnp.stack([lint_b, lins_b])

    degs = _sc_degrees(sidx_all)
    uh, g0, scv = _mlp(u0, wcat, bcat, degs.reshape(NC * NP, 1))

    acc1 = _sc_round(g0, gidx_all, sidx_all)
    cv0 = conv_w[0, ::-1].reshape(2, 1, 1)
    u1, g1 = _combine(uh, acc1, scv, cv0, True)

    acc2 = _sc_round(g1, gidx_all, sidx_all)
    cv1 = conv_w[1, ::-1].reshape(2, 1, 1)
    u2 = _combine(u1, acc2, scv, cv1, False)[0]

    return (u2[NP:NP + N], u2[:N])


# speedup vs baseline: 18.9530x; 1.1014x over previous
import functools

import jax
import jax.numpy as jnp
from jax import lax
from jax.experimental import pallas as pl
from jax.experimental.pallas import tpu as pltpu, tpu_sc as plsc

N = 10000
D = 128
NP = 10240
NPAD_ROWS = 8
NC, NS, L = 2, 16, 16
CB = 128
C = 160
EPT = C * CB
EP = NS * EPT
RPT = NP // NS

_mesh = plsc.VectorSubcoreMesh(
    core_axis_name="c", subcore_axis_name="s", num_cores=NC, num_subcores=NS)


@functools.partial(
    pl.kernel,
    out_type=jax.ShapeDtypeStruct((NC * NP,), jnp.float32),
    mesh=_mesh,
    scratch_types=[
        pltpu.VMEM((C, CB), jnp.int32),
        pltpu.VMEM((CB,), jnp.float32),
        pltpu.VMEM((RPT,), jnp.float32),
        pltpu.VMEM_SHARED((NP,), jnp.float32),
        pltpu.SemaphoreType.DMA,
    ],
)
def _sc_degrees(sidx_all, degs, idx_v, ones_v, z_v, hist, sem):
    cid = lax.axis_index("c")
    sid = lax.axis_index("s")
    for i in range(CB // L):
        ones_v[pl.ds(i * L, L)] = jnp.ones((L,), jnp.float32)
    for i in range(RPT // L):
        z_v[pl.ds(i * L, L)] = jnp.zeros((L,), jnp.float32)
    pltpu.sync_copy(z_v, hist.at[pl.ds(sid * RPT, RPT)])
    pltpu.sync_copy(sidx_all.at[pl.ds((cid * NS + sid) * C, C)], idx_v)
    plsc.subcore_barrier()

    @pl.loop(0, C)
    def _(c):
        pltpu.sync_copy(ones_v, hist.at[idx_v.at[c]], add=True)

    plsc.subcore_barrier()
    pltpu.sync_copy(hist.at[pl.ds(sid * RPT, RPT)],
                    degs.at[pl.ds((1 - cid) * NP + sid * RPT, RPT)])


HD = D // 2


@functools.partial(
    pl.kernel,
    out_type=jax.ShapeDtypeStruct((2, NC * NP, HD), jnp.float32),
    mesh=_mesh,
    scratch_types=[
        pltpu.VMEM((C, CB), jnp.int32),
        pltpu.VMEM((C, CB), jnp.int32),
        pltpu.VMEM((CB, HD), jnp.float32),
        pltpu.VMEM((CB, HD), jnp.float32),
        pltpu.VMEM((CB, HD), jnp.float32),
        pltpu.VMEM((CB, HD), jnp.float32),
        pltpu.VMEM_SHARED((NP, HD), jnp.float32),
        pltpu.SemaphoreType.DMA,
        pltpu.SemaphoreType.DMA,
        pltpu.SemaphoreType.DMA,
        pltpu.SemaphoreType.DMA,
    ],
    compiler_params=pltpu.CompilerParams(use_tc_tiling_on_sc=False),
)
def _sc_round(g2, gidx_all, sidx_all, acc2, gidx, sidx,
              rows0, rows1, rows2, rows3, acc, sem0, sem1, sem2, sem3):
    cid = lax.axis_index("c")
    sid = lax.axis_index("s")
    rsl = pl.ds(sid * RPT, RPT)
    csl = pl.ds((cid * NS + sid) * C, C)

    pltpu.sync_copy(gidx_all.at[csl], gidx)
    pltpu.sync_copy(sidx_all.at[csl], sidx)

    for hh in range(2):
        src = g2.at[hh]
        pltpu.sync_copy(src.at[pl.ds(cid * NP + sid * RPT, RPT)],
                        acc.at[rsl])
        plsc.subcore_barrier()

        bufs = (rows0, rows1, rows2, rows3)
        sems = (sem0, sem1, sem2, sem3)
        nb = len(bufs)
        for b in range(nb - 1):
            pltpu.async_copy(src.at[gidx.at[b]], bufs[b], sems[b])

        @pl.loop(0, C, step=nb)
        def _(c):
            for b in range(nb):
                pre = c + b + nb - 1
                pb = (b + nb - 1) % nb

                @pl.when(pre < C)
                def _():
                    pltpu.async_copy(src.at[gidx.at[pre]], bufs[pb],
                                     sems[pb])

                pltpu.make_async_copy(src.at[pl.ds(0, CB)],
                                      bufs[b], sems[b]).wait()
                pltpu.sync_copy(bufs[b], acc.at[sidx.at[c + b]], add=True)

        plsc.subcore_barrier()
        pltpu.sync_copy(
            acc.at[rsl],
            acc2.at[hh].at[pl.ds((1 - cid) * NP + sid * RPT, RPT)])
        plsc.subcore_barrier()


_BR = 1280
_HGRID = NP // _BR
_GRID = NC * _HGRID

_dn = (((1,), (1,)), ((), ()))


def _dot_t(x, w):
    return lax.dot_general(x, w, dimension_numbers=_dn,
                           preferred_element_type=jnp.float32,
                           precision=lax.Precision.HIGHEST)


def _rows_spec():
    return pl.BlockSpec((_BR, D), lambda i: (i, 0))


def _colv_spec():
    return pl.BlockSpec((_BR, 1), lambda i: (i, 0))


def _gsplit_spec():
    return pl.BlockSpec((2, _BR, HD), lambda i: (0, i, 0))


def _write_split(g_ref, full):
    g_ref[0] = full[:, :HD]
    g_ref[1] = full[:, HD:]


def _mlp_body(u_ref, w_ref, b_ref, deg_ref, uh_ref, g_ref, sc_ref):
    x = u_ref[...]
    h = jnp.maximum(_dot_t(x, w_ref[0, 0]) + b_ref[0, 0][None, :], 0.0)
    xh = _dot_t(h, w_ref[0, 1]) + b_ref[0, 1][None, :]
    sc = lax.rsqrt(deg_ref[...] + 1.0)
    uh_ref[...] = xh
    sc_ref[...] = sc
    _write_split(g_ref, sc * xh)


def _mlp(u, wcat, bcat, deg2):
    out_nd = jax.ShapeDtypeStruct((NC * NP, D), jnp.float32)
    out_g = jax.ShapeDtypeStruct((2, NC * NP, HD), jnp.float32)
    out_n1 = jax.ShapeDtypeStruct((NC * NP, 1), jnp.float32)
    return pl.pallas_call(
        _mlp_body,
        grid=(_GRID,),
        in_specs=[
            _rows_spec(),
            pl.BlockSpec((1, 2, D, D), lambda i: (i // _HGRID, 0, 0, 0)),
            pl.BlockSpec((1, 2, D), lambda i: (i // _HGRID, 0, 0)),
            _colv_spec(),
        ],
        out_specs=[_rows_spec(), _gsplit_spec(), _colv_spec()],
        out_shape=[out_nd, out_g, out_n1],
    )(u, wcat, bcat, deg2)


def _combine_body(with_g, u_ref, acc_ref, sc_ref, cw_ref, *outs):
    sc = sc_ref[...]
    accf = jnp.concatenate([acc_ref[0], acc_ref[1]], axis=1)
    u1 = u_ref[...] + cw_ref[0, 0, 0] * sc * accf
    outs[0][...] = u1
    if with_g:
        _write_split(outs[1], sc * u1)


def _combine(u, acc2, scv, cvec, with_g):
    out_nd = jax.ShapeDtypeStruct((NC * NP, D), jnp.float32)
    out_g = jax.ShapeDtypeStruct((2, NC * NP, HD), jnp.float32)
    out_specs = [_rows_spec()]
    out_shape = [out_nd]
    if with_g:
        out_specs.append(_gsplit_spec())
        out_shape.append(out_g)
    return pl.pallas_call(
        functools.partial(_combine_body, with_g),
        grid=(_GRID,),
        in_specs=[
            _rows_spec(),
            _gsplit_spec(),
            _colv_spec(),
            pl.BlockSpec((1, 1, 1), lambda i: (i // _HGRID, 0, 0)),
        ],
        out_specs=out_specs,
        out_shape=out_shape,
    )(u, acc2, scv, cvec)


@jax.jit
def kernel(s, t, edge_index, lins_W, lins_b, lint_W, lint_b, conv_w):
    row = edge_index[0]
    col = edge_index[1]
    e = row.shape[0]
    pad = EP - e
    padv = N + (jnp.arange(pad, dtype=jnp.int32) % NPAD_ROWS)
    rowp = jnp.concatenate([row, padv])
    colp = jnp.concatenate([col, padv])
    gidx_all = jnp.concatenate([colp, rowp + NP]).reshape(NC * NS * C, CB)
    sidx_all = jnp.concatenate([rowp, colp]).reshape(NC * NS * C, CB)

    zpad = jnp.zeros((NP - N, D), jnp.float32)
    u0 = jnp.concatenate([t, zpad, s, zpad])

    wcat = jnp.stack([lint_W, lins_W])
    bcat = jnp.stack([lint_b, lins_b])

    degs = _sc_degrees(sidx_all)
    uh, g0, scv = _mlp(u0, wcat, bcat, degs.reshape(NC * NP, 1))

    acc1 = _sc_round(g0, gidx_all, sidx_all)
    cv0 = conv_w[0, ::-1].reshape(2, 1, 1)
    u1, g1 = _combine(uh, acc1, scv, cv0, True)

    acc2 = _sc_round(g1, gidx_all, sidx_all)
    cv1 = conv_w[1, ::-1].reshape(2, 1, 1)
    u2 = _combine(u1, acc2, scv, cv1, False)[0]

    return (u2[NP:NP + N], u2[:N])
